# baseline (device time: 68700 ns/iter reference)
import jax
import jax.numpy as jnp
from jax import lax
from jax.experimental import pallas as pl
from jax.experimental.pallas import tpu as pltpu

N_DEV = 4
B = 2
S_LOC = 512
D = 1024
HQ = 8
DH = 128
SCALE = 0.08838834764831843
ROWS = B * S_LOC


def _body(x_ref, wq_hbm, wk_hbm, wv_hbm, wo_hbm, out_ref,
          comm_r, comm_l, q_ref, stage,
          send_r, recv_r, send_l, recv_l, w_sem):
    my = lax.axis_index("i")
    left = lax.rem(my + N_DEV - 1, N_DEV)
    right = lax.rem(my + 1, N_DEV)

    barrier = pltpu.get_barrier_semaphore()
    for nbr in (left, right):
        pl.semaphore_signal(barrier, inc=1, device_id=(nbr,),
                            device_id_type=pl.DeviceIdType.MESH)
    pl.semaphore_wait(barrier, 2)

    xb = x_ref[...].astype(jnp.bfloat16)
    comm_r[0] = xb[0:S_LOC, :]
    comm_l[0] = xb[S_LOC:ROWS, :]

    def make_hop(h):
        rr = pltpu.make_async_remote_copy(
            src_ref=comm_r.at[h], dst_ref=comm_r.at[h + 1],
            send_sem=send_r.at[h], recv_sem=recv_r.at[h],
            device_id=(right,), device_id_type=pl.DeviceIdType.MESH)
        rl = pltpu.make_async_remote_copy(
            src_ref=comm_l.at[h], dst_ref=comm_l.at[h + 1],
            send_sem=send_l.at[h], recv_sem=recv_l.at[h],
            device_id=(left,), device_id_type=pl.DeviceIdType.MESH)
        rr.start()
        rl.start()
        return rr, rl

    hop0 = make_hop(0)

    def fetch_w(w_hbm, scale=None):
        cp = pltpu.make_async_copy(w_hbm, stage, w_sem)
        cp.start()
        cp.wait()
        w = stage[...]
        if scale is not None:
            w = w * scale
        return w.astype(jnp.bfloat16)

    wq_b = fetch_w(wq_hbm, SCALE)
    wk_b = fetch_w(wk_hbm)
    wv_b = fetch_w(wv_hbm)
    wkv_b = jnp.concatenate([wk_b, wv_b], axis=1)

    lj = lax.broadcasted_iota(jnp.int32, (S_LOC, DH), 1)
    i2 = (lj - lax.rem(lj, 2)).astype(jnp.float32)
    inv = jnp.exp(i2 * (-jnp.log(10000.0) / DH))
    s_iota = lax.broadcasted_iota(jnp.int32, (S_LOC, DH), 0).astype(jnp.float32)
    theta = s_iota * inv
    cos_s = jnp.cos(theta).astype(jnp.bfloat16)
    sin_s = jnp.sin(theta).astype(jnp.bfloat16)
    even = lax.rem(lax.broadcasted_iota(jnp.int32, (S_LOC, D), 1), 2) == 0

    _tables = {}

    def rope_tables(d):
        if d not in _tables:
            origin = lax.rem(my + d, N_DEV)
            off = (origin * S_LOC).astype(jnp.float32)
            th_o = off * inv[0:1, :]
            cos_o = jnp.cos(th_o).astype(jnp.bfloat16)
            sin_o = jnp.sin(th_o).astype(jnp.bfloat16)
            cos_f = cos_o * cos_s - sin_o * sin_s
            sin_f = sin_o * cos_s + cos_o * sin_s
            _tables[d] = (jnp.concatenate([cos_f] * HQ, axis=1),
                          jnp.concatenate([sin_f] * HQ, axis=1))
        return _tables[d]

    def apply_rope_half(t, cos_f, sin_f):
        t_next = pltpu.roll(t, D - 1, 1)
        t_prev = pltpu.roll(t, 1, 1)
        t_rot = jnp.where(even, -t_next, t_prev)
        return t * cos_f + t_rot * sin_f

    cos_my, sin_my = rope_tables(0)
    xq = jnp.dot(xb, wq_b,
                 preferred_element_type=jnp.float32).astype(jnp.bfloat16)
    q_ref[0:S_LOC, :] = apply_rope_half(xq[0:S_LOC, :], cos_my, sin_my)
    q_ref[S_LOC:ROWS, :] = apply_rope_half(xq[S_LOC:ROWS, :], cos_my, sin_my)

    l_st = {}
    acc = {}

    def attn_half(k, kk, vv, b):
        r0 = b * S_LOC
        for h in range(HQ):
            c0, c1 = h * DH, (h + 1) * DH
            q_bh = q_ref[r0:r0 + S_LOC, c0:c1]
            s = lax.dot_general(
                q_bh, kk[:, c0:c1], (((1,), (1,)), ((), ())),
                preferred_element_type=jnp.float32)
            p = jnp.exp(s)
            if k == 0:
                l_st[b, h] = jnp.sum(p, axis=1, keepdims=True)
                acc[b, h] = jnp.dot(
                    p.astype(jnp.bfloat16), vv[:, c0:c1],
                    preferred_element_type=jnp.float32)
            else:
                l_st[b, h] = l_st[b, h] + jnp.sum(p, axis=1, keepdims=True)
                acc[b, h] = acc[b, h] + jnp.dot(
                    p.astype(jnp.bfloat16), vv[:, c0:c1],
                    preferred_element_type=jnp.float32)

    def process_pair(h, xcat):
        xkv = jnp.dot(xcat, wkv_b,
                      preferred_element_type=jnp.float32).astype(jnp.bfloat16)
        xk = xkv[:, 0:D]
        xv = xkv[:, D:2 * D]
        cr, sr = rope_tables(N_DEV - h if h else 0)
        cl, sl = rope_tables(h)
        attn_half(h, apply_rope_half(xk[0:S_LOC, :], cr, sr),
                  xv[0:S_LOC, :], 0)
        attn_half(h, apply_rope_half(xk[S_LOC:ROWS, :], cl, sl),
                  xv[S_LOC:ROWS, :], 1)

    for h in range(N_DEV):
        if h == 0:
            rr, rl = hop0
        elif h < N_DEV - 1:
            rr, rl = make_hop(h)
        else:
            rr = rl = None
        if h == 0:
            xcat = xb
        else:
            xcat = jnp.concatenate([comm_r[h], comm_l[h]], axis=0)
        process_pair(h, xcat)
        if rr is not None:
            rr.wait()
            rl.wait()

    for b in range(B):
        r0 = b * S_LOC
        for h in range(HQ):
            c0, c1 = h * DH, (h + 1) * DH
            q_ref[r0:r0 + S_LOC, c0:c1] = (
                acc[b, h] * (1.0 / l_st[b, h])).astype(jnp.bfloat16)

    wo_b = fetch_w(wo_hbm)
    out_ref[...] = jnp.dot(q_ref[...], wo_b,
                           preferred_element_type=jnp.float32).astype(jnp.bfloat16)


def kernel(x, Wq, Wk, Wv, Wo):
    out2d = pl.pallas_call(
        _body,
        out_shape=jax.ShapeDtypeStruct((ROWS, D), jnp.bfloat16),
        in_specs=[pl.BlockSpec(memory_space=pltpu.VMEM)]
        + [pl.BlockSpec(memory_space=pl.ANY)] * 4,
        out_specs=pl.BlockSpec(memory_space=pltpu.VMEM),
        scratch_shapes=[
            pltpu.VMEM((N_DEV, S_LOC, D), jnp.bfloat16),
            pltpu.VMEM((N_DEV, S_LOC, D), jnp.bfloat16),
            pltpu.VMEM((ROWS, D), jnp.bfloat16),
            pltpu.VMEM((D, D), jnp.float32),
            pltpu.SemaphoreType.DMA((N_DEV - 1,)),
            pltpu.SemaphoreType.DMA((N_DEV - 1,)),
            pltpu.SemaphoreType.DMA((N_DEV - 1,)),
            pltpu.SemaphoreType.DMA((N_DEV - 1,)),
            pltpu.SemaphoreType.DMA,
        ],
        compiler_params=pltpu.CompilerParams(
            collective_id=0,
            vmem_limit_bytes=63 * 1024 * 1024,
        ),
    )(x.reshape(ROWS, D), Wq, Wk, Wv, Wo)
    return out2d.reshape(B, S_LOC, D)
